# baseline (speedup 1.0000x reference)
import functools

import jax
import jax.numpy as jnp
from jax import lax
from jax.experimental import pallas as pl
from jax.experimental.pallas import tpu as pltpu
from jax.experimental.pallas import tpu_sc as plsc

N = 50000
M = 16
ORIG = 92
NBR = 16
AF = 64
HF = 128
NCONV = 3
N0 = 500
APC = 100
NM = N * M
GF = 2 * AF


_CHI = 1000
_CG0 = 504
_CG1 = 496


def _sc_gather(table, idx_flat):
    info = plsc.get_sparse_core_info()
    nc, ns = info.num_cores, info.num_subcores
    nw = nc * ns
    nmh = NM // 2
    nsup = 12

    mesh = plsc.VectorSubcoreMesh(core_axis_name="c", subcore_axis_name="s")

    @functools.partial(
        pl.kernel,
        mesh=mesh,
        out_type=jax.ShapeDtypeStruct((nmh, GF), jnp.float32),
        scratch_types=[
            pltpu.VMEM((_CHI,), jnp.int32),
            pltpu.VMEM((_CHI,), jnp.int32),
            pltpu.VMEM((_CG0, GF), jnp.float32),
            pltpu.VMEM((_CG1, GF), jnp.float32),
            pltpu.SemaphoreType.DMA,
            pltpu.SemaphoreType.DMA,
        ],
    )
    def gather_k(tab_hbm, idx_hbm, out_hbm, idx0, idx1, rows0, rows1,
                 sem0, sem1):
        wid = lax.axis_index("s") * nc + lax.axis_index("c")
        base = pl.multiple_of(wid * (nmh // nw) - 4 * lax.rem(wid, 2), 8)
        idxs = (idx0, idx1)

        def gstart0(iv):
            return pltpu.async_copy(tab_hbm.at[iv.at[pl.ds(0, _CG0)]],
                                    rows0, sem0)

        def gstart1(iv):
            return pltpu.async_copy(tab_hbm.at[iv.at[pl.ds(_CG0, _CG1)]],
                                    rows1, sem1)

        pltpu.sync_copy(idx_hbm.at[pl.ds(base, _CHI)], idx0)
        cp = gstart0(idx0)
        for c in range(nsup):
            iv = idxs[c % 2]
            cp2 = gstart1(iv)
            cp.wait()
            pltpu.sync_copy(rows0, out_hbm.at[pl.ds(base + c * _CHI, _CG0)])
            if c + 1 < nsup:
                nxt = idxs[(c + 1) % 2]
                pltpu.sync_copy(
                    idx_hbm.at[pl.ds(base + (c + 1) * _CHI, _CHI)], nxt)
                cp = gstart0(nxt)
            cp2.wait()
            pltpu.sync_copy(
                rows1, out_hbm.at[pl.ds(base + c * _CHI + _CG0, _CG1)])
        toff = base + nsup * _CHI
        pltpu.sync_copy(idx_hbm.at[pl.ds(toff, _CG0)],
                        idx0.at[pl.ds(0, _CG0)])
        cp = gstart0(idx0)
        cp.wait()
        pltpu.sync_copy(rows0, out_hbm.at[pl.ds(toff, _CG0)])

    return gather_k(table, idx_flat)


_BR = 2000
_BA = 1000
_BE = _BA * M


def _embed(atom_fea, w_t, b):
    def body(a_ref, w_ref, b_ref, o_ref):
        o_ref[...] = (
            jnp.dot(a_ref[...], w_ref[...], preferred_element_type=jnp.float32)
            + b_ref[...]
        )

    return pl.pallas_call(
        body,
        grid=(N // _BR,),
        in_specs=[
            pl.BlockSpec((_BR, ORIG), lambda i: (i, 0)),
            pl.BlockSpec((ORIG, AF), lambda i: (0, 0)),
            pl.BlockSpec((1, AF), lambda i: (0, 0)),
        ],
        out_specs=pl.BlockSpec((_BR, AF), lambda i: (i, 0)),
        out_shape=jax.ShapeDtypeStruct((N, AF), jnp.float32),
    )(atom_fea, w_t, b)


def _pre(x, ws_t, wn_t, bfi):
    def body(x_ref, ws_ref, wn_ref, bf_ref, a_ref, b_ref):
        xb = x_ref[...]
        a_ref[...] = (
            jnp.dot(xb, ws_ref[...], preferred_element_type=jnp.float32)
            + bf_ref[...]
        )
        b_ref[...] = jnp.dot(xb, wn_ref[...], preferred_element_type=jnp.float32)

    return pl.pallas_call(
        body,
        grid=(N // _BR,),
        in_specs=[
            pl.BlockSpec((_BR, AF), lambda i: (i, 0)),
            pl.BlockSpec((AF, GF), lambda i: (0, 0)),
            pl.BlockSpec((AF, GF), lambda i: (0, 0)),
            pl.BlockSpec((1, GF), lambda i: (0, 0)),
        ],
        out_specs=[
            pl.BlockSpec((_BR, GF), lambda i: (i, 0)),
            pl.BlockSpec((_BR, GF), lambda i: (i, 0)),
        ],
        out_shape=[
            jax.ShapeDtypeStruct((N, GF), jnp.float32),
            jax.ShapeDtypeStruct((N, GF), jnp.float32),
        ],
    )(x, ws_t, wn_t, bfi)


_BA2 = 1024
_NBLK = -(-N // _BA2)


def _valid_mask(pid):
    row = lax.broadcasted_iota(jnp.int32, (_BA2, 1), 0)
    return row < (N - pid * _BA2)


_MH = M // 2


def _pass1h(a, nbh, e_nat, we_t, h):
    def body(a_ref, nb_ref, e_ref, we_ref, g_ref, s_ref, q_ref):
        a2 = a_ref[...]
        s_rows = jnp.zeros((_BA2, GF), jnp.float32)
        q_rows = jnp.zeros((_BA2, GF), jnp.float32)
        for m in range(_MH):
            ecm = lax.dot_general(
                e_ref[m], we_ref[...],
                dimension_numbers=(((0,), (0,)), ((), ())),
                preferred_element_type=jnp.float32,
            )
            gm = nb_ref[m] + ecm + a2
            g_ref[m] = gm.astype(jnp.bfloat16)
            s_rows = s_rows + gm
            q_rows = q_rows + gm * gm

        mask = _valid_mask(pl.program_id(0))
        s_rows = jnp.where(mask, s_rows, 0.0)
        q_rows = jnp.where(mask, q_rows, 0.0)

        @pl.when(pl.program_id(0) == 0)
        def _():
            s_ref[...] = jnp.zeros_like(s_ref)
            q_ref[...] = jnp.zeros_like(q_ref)

        s_ref[...] += jnp.sum(s_rows, axis=0).reshape(1, GF)
        q_ref[...] += jnp.sum(q_rows, axis=0).reshape(1, GF)

    return pl.pallas_call(
        body,
        grid=(_NBLK,),
        in_specs=[
            pl.BlockSpec((_BA2, GF), lambda i: (i, 0)),
            pl.BlockSpec((_MH, _BA2, GF), lambda i: (0, i, 0)),
            pl.BlockSpec((_MH, NBR, _BA2), lambda i: (h, 0, i)),
            pl.BlockSpec((NBR, GF), lambda i: (0, 0)),
        ],
        out_specs=[
            pl.BlockSpec((_MH, _BA2, GF), lambda i: (0, i, 0)),
            pl.BlockSpec((1, GF), lambda i: (0, 0)),
            pl.BlockSpec((1, GF), lambda i: (0, 0)),
        ],
        out_shape=[
            jax.ShapeDtypeStruct((_MH, N, GF), jnp.bfloat16),
            jax.ShapeDtypeStruct((1, GF), jnp.float32),
            jax.ShapeDtypeStruct((1, GF), jnp.float32),
        ],
    )(a, nbh, e_nat, we_t)


def _pass2(g1, g2, sc1, sh1):
    def body(g1_ref, g2_ref, sc_ref, sh_ref, sum_ref, s_ref, q_ref):
        sm = jnp.zeros((_BA2, AF), jnp.float32)
        for g_ref in (g1_ref, g2_ref):
            y = g_ref[...].astype(jnp.float32) * sc_ref[...] + sh_ref[...]
            u = y[..., :AF]
            v = y[..., AF:]
            filt = 0.5 * jnp.tanh(0.5 * u) + 0.5
            core = jnp.where(v > 30.0, v, jnp.log1p(jnp.exp(v)))
            sm = sm + jnp.sum(filt * core, axis=0)
        sum_ref[...] = sm

        mask = _valid_mask(pl.program_id(0))
        smm = jnp.where(mask, sm, 0.0)

        @pl.when(pl.program_id(0) == 0)
        def _():
            s_ref[...] = jnp.zeros_like(s_ref)
            q_ref[...] = jnp.zeros_like(q_ref)

        s_ref[...] += jnp.sum(smm, axis=0).reshape(1, AF)
        q_ref[...] += jnp.sum(smm * smm, axis=0).reshape(1, AF)

    return pl.pallas_call(
        body,
        grid=(_NBLK,),
        in_specs=[
            pl.BlockSpec((_MH, _BA2, GF), lambda i: (0, i, 0)),
            pl.BlockSpec((_MH, _BA2, GF), lambda i: (0, i, 0)),
            pl.BlockSpec((1, GF), lambda i: (0, 0)),
            pl.BlockSpec((1, GF), lambda i: (0, 0)),
        ],
        out_specs=[
            pl.BlockSpec((_BA2, AF), lambda i: (i, 0)),
            pl.BlockSpec((1, AF), lambda i: (0, 0)),
            pl.BlockSpec((1, AF), lambda i: (0, 0)),
        ],
        out_shape=[
            jax.ShapeDtypeStruct((N, AF), jnp.float32),
            jax.ShapeDtypeStruct((1, AF), jnp.float32),
            jax.ShapeDtypeStruct((1, AF), jnp.float32),
        ],
    )(g1, g2, sc1, sh1)


def _pass3(x, sm, sc2, sh2):
    def body(x_ref, sm_ref, sc_ref, sh_ref, o_ref):
        t = x_ref[...] + sm_ref[...] * sc_ref[...] + sh_ref[...]
        o_ref[...] = jax.nn.softplus(t)

    return pl.pallas_call(
        body,
        grid=(N // _BR,),
        in_specs=[
            pl.BlockSpec((_BR, AF), lambda i: (i, 0)),
            pl.BlockSpec((_BR, AF), lambda i: (i, 0)),
            pl.BlockSpec((1, AF), lambda i: (0, 0)),
            pl.BlockSpec((1, AF), lambda i: (0, 0)),
        ],
        out_specs=pl.BlockSpec((_BR, AF), lambda i: (i, 0)),
        out_shape=jax.ShapeDtypeStruct((N, AF), jnp.float32),
    )(x, sm, sc2, sh2)


def _head(x3, wfc_t, bfc, wh_t, bh, wo_t, bo):
    def body(x_ref, wfc_ref, bfc_ref, wh_ref, bh_ref, wo_ref, bo_ref, o_ref):
        crys = jnp.mean(x_ref[...], axis=1)
        crys = jax.nn.relu(
            jnp.dot(crys, wfc_ref[...], preferred_element_type=jnp.float32)
            + bfc_ref[...]
        )
        h = (
            jnp.dot(crys, wh_ref[...], preferred_element_type=jnp.float32)
            + bh_ref[...]
        )
        o_ref[...] = (
            jnp.dot(h, wo_ref[...], preferred_element_type=jnp.float32)
            + bo_ref[...]
        )

    return pl.pallas_call(
        body,
        in_specs=[
            pl.BlockSpec((N0, APC, AF), lambda: (0, 0, 0)),
            pl.BlockSpec((AF, HF), lambda: (0, 0)),
            pl.BlockSpec((1, HF), lambda: (0, 0)),
            pl.BlockSpec((HF, 256), lambda: (0, 0)),
            pl.BlockSpec((1, 256), lambda: (0, 0)),
            pl.BlockSpec((256, 1), lambda: (0, 0)),
            pl.BlockSpec((1, 1), lambda: (0, 0)),
        ],
        out_specs=pl.BlockSpec((N0, 1), lambda: (0, 0)),
        out_shape=jax.ShapeDtypeStruct((N0, 1), jnp.float32),
    )(x3, wfc_t, bfc, wh_t, bh, wo_t, bo)


def kernel(atom_fea, nbr_fea, nbr_fea_idx, crystal_atom_idx,
           W_emb, b_emb, Wf, bf, g1, bb1, g2, bb2,
           W_fc, b_fc, W_h, b_h, W_o, b_o):
    idx2 = nbr_fea_idx.T.reshape(2, NM // 2)
    e_nat = jnp.transpose(nbr_fea, (1, 2, 0))
    x = _embed(atom_fea, W_emb.T, b_emb.reshape(1, AF))

    for i in range(NCONV):
        ws_t = Wf[i, :, :AF].T
        wn_t = Wf[i, :, AF:2 * AF].T
        we_t = Wf[i, :, 2 * AF:].T
        bfi = bf[i].reshape(1, GF)

        a, b_tab = _pre(x, ws_t, wn_t, bfi)
        nbh1 = _sc_gather(b_tab, idx2[0])
        nbh2 = _sc_gather(b_tab, idx2[1])

        gs1, s1a, q1a = _pass1h(a, nbh1.reshape(_MH, N, GF), e_nat, we_t, 0)
        gs2, s1b, q1b = _pass1h(a, nbh2.reshape(_MH, N, GF), e_nat, we_t, 1)
        s1 = s1a + s1b
        q1 = q1a + q1b
        mean1 = s1 / NM
        var1 = q1 / NM - mean1 * mean1
        sc1 = g1[i].reshape(1, GF) / jnp.sqrt(var1 + 1e-5)
        sh1 = bb1[i].reshape(1, GF) - mean1 * sc1

        sm, s2, q2 = _pass2(gs1, gs2, sc1, sh1)
        mean2 = s2 / N
        var2 = q2 / N - mean2 * mean2
        sc2 = g2[i].reshape(1, AF) / jnp.sqrt(var2 + 1e-5)
        sh2 = bb2[i].reshape(1, AF) - mean2 * sc2

        x = _pass3(x, sm, sc2, sh2)

    out = _head(x.reshape(N0, APC, AF), W_fc.T, b_fc.reshape(1, HF),
                W_h.T, b_h.reshape(1, 256), W_o.T, b_o.reshape(1, 1))
    return (out,)


# speedup vs baseline: 1.0244x; 1.0244x over previous
import functools

import jax
import jax.numpy as jnp
from jax import lax
from jax.experimental import pallas as pl
from jax.experimental.pallas import tpu as pltpu
from jax.experimental.pallas import tpu_sc as plsc

N = 50000
M = 16
ORIG = 92
NBR = 16
AF = 64
HF = 128
NCONV = 3
N0 = 500
APC = 100
NM = N * M
GF = 2 * AF


_CHI = 1000
_CG0 = 504
_CG1 = 496


def _sc_gather(table, idx_flat):
    info = plsc.get_sparse_core_info()
    nc, ns = info.num_cores, info.num_subcores
    nw = nc * ns
    epw = NM // nw
    nsup = epw // _CHI

    mesh = plsc.VectorSubcoreMesh(core_axis_name="c", subcore_axis_name="s")

    @functools.partial(
        pl.kernel,
        mesh=mesh,
        out_type=jax.ShapeDtypeStruct((NM, GF), jnp.float32),
        scratch_types=[
            pltpu.VMEM((_CHI,), jnp.int32),
            pltpu.VMEM((_CHI,), jnp.int32),
            pltpu.VMEM((_CG0, GF), jnp.float32),
            pltpu.VMEM((_CG1, GF), jnp.float32),
            pltpu.SemaphoreType.DMA,
            pltpu.SemaphoreType.DMA,
        ],
    )
    def gather_k(tab_hbm, idx_hbm, out_hbm, idx0, idx1, rows0, rows1,
                 sem0, sem1):
        wid = lax.axis_index("s") * nc + lax.axis_index("c")
        base = wid * epw
        idxs = (idx0, idx1)

        def gstart0(iv):
            return pltpu.async_copy(tab_hbm.at[iv.at[pl.ds(0, _CG0)]],
                                    rows0, sem0)

        def gstart1(iv):
            return pltpu.async_copy(tab_hbm.at[iv.at[pl.ds(_CG0, _CG1)]],
                                    rows1, sem1)

        pltpu.sync_copy(idx_hbm.at[pl.ds(base, _CHI)], idx0)
        cp = gstart0(idx0)
        for c in range(nsup):
            iv = idxs[c % 2]
            cp2 = gstart1(iv)
            cp.wait()
            pltpu.sync_copy(rows0, out_hbm.at[pl.ds(base + c * _CHI, _CG0)])
            if c + 1 < nsup:
                nxt = idxs[(c + 1) % 2]
                pltpu.sync_copy(
                    idx_hbm.at[pl.ds(base + (c + 1) * _CHI, _CHI)], nxt)
                cp = gstart0(nxt)
            cp2.wait()
            pltpu.sync_copy(
                rows1, out_hbm.at[pl.ds(base + c * _CHI + _CG0, _CG1)])

    return gather_k(table, idx_flat)


_BR = 2000
_BA = 1000
_BE = _BA * M


def _embed(atom_fea, w_t, b):
    def body(a_ref, w_ref, b_ref, o_ref):
        o_ref[...] = (
            jnp.dot(a_ref[...], w_ref[...], preferred_element_type=jnp.float32)
            + b_ref[...]
        )

    return pl.pallas_call(
        body,
        grid=(N // _BR,),
        in_specs=[
            pl.BlockSpec((_BR, ORIG), lambda i: (i, 0)),
            pl.BlockSpec((ORIG, AF), lambda i: (0, 0)),
            pl.BlockSpec((1, AF), lambda i: (0, 0)),
        ],
        out_specs=pl.BlockSpec((_BR, AF), lambda i: (i, 0)),
        out_shape=jax.ShapeDtypeStruct((N, AF), jnp.float32),
    )(atom_fea, w_t, b)


def _pre(x, ws_t, wn_t, bfi):
    def body(x_ref, ws_ref, wn_ref, bf_ref, a_ref, b_ref):
        xb = x_ref[...]
        a_ref[...] = (
            jnp.dot(xb, ws_ref[...], preferred_element_type=jnp.float32)
            + bf_ref[...]
        )
        b_ref[...] = jnp.dot(xb, wn_ref[...], preferred_element_type=jnp.float32)

    return pl.pallas_call(
        body,
        grid=(N // _BR,),
        in_specs=[
            pl.BlockSpec((_BR, AF), lambda i: (i, 0)),
            pl.BlockSpec((AF, GF), lambda i: (0, 0)),
            pl.BlockSpec((AF, GF), lambda i: (0, 0)),
            pl.BlockSpec((1, GF), lambda i: (0, 0)),
        ],
        out_specs=[
            pl.BlockSpec((_BR, GF), lambda i: (i, 0)),
            pl.BlockSpec((_BR, GF), lambda i: (i, 0)),
        ],
        out_shape=[
            jax.ShapeDtypeStruct((N, GF), jnp.float32),
            jax.ShapeDtypeStruct((N, GF), jnp.float32),
        ],
    )(x, ws_t, wn_t, bfi)


_BA2 = 1024
_NBLK = -(-N // _BA2)


def _valid_mask(pid):
    row = lax.broadcasted_iota(jnp.int32, (_BA2, 1), 0)
    return row < (N - pid * _BA2)


_MH = M // 2


def _pass1(a, nb3, e_nat, we_t):
    def body(a_ref, nb_ref, e_ref, we_ref, g0_ref, g1_ref, s_ref, q_ref):
        a2 = a_ref[...]
        s_rows = jnp.zeros((_BA2, GF), jnp.float32)
        q_rows = jnp.zeros((_BA2, GF), jnp.float32)
        for m in range(M):
            ecm = lax.dot_general(
                e_ref[m], we_ref[...],
                dimension_numbers=(((0,), (0,)), ((), ())),
                preferred_element_type=jnp.float32,
            )
            gm = nb_ref[m] + ecm + a2
            if m < _MH:
                g0_ref[m] = gm.astype(jnp.bfloat16)
            else:
                g1_ref[m - _MH] = gm.astype(jnp.bfloat16)
            s_rows = s_rows + gm
            q_rows = q_rows + gm * gm

        mask = _valid_mask(pl.program_id(0))
        s_rows = jnp.where(mask, s_rows, 0.0)
        q_rows = jnp.where(mask, q_rows, 0.0)

        @pl.when(pl.program_id(0) == 0)
        def _():
            s_ref[...] = jnp.zeros_like(s_ref)
            q_ref[...] = jnp.zeros_like(q_ref)

        s_ref[...] += jnp.sum(s_rows, axis=0).reshape(1, GF)
        q_ref[...] += jnp.sum(q_rows, axis=0).reshape(1, GF)

    return pl.pallas_call(
        body,
        grid=(_NBLK,),
        in_specs=[
            pl.BlockSpec((_BA2, GF), lambda i: (i, 0)),
            pl.BlockSpec((M, _BA2, GF), lambda i: (0, i, 0)),
            pl.BlockSpec((M, NBR, _BA2), lambda i: (0, 0, i)),
            pl.BlockSpec((NBR, GF), lambda i: (0, 0)),
        ],
        out_specs=[
            pl.BlockSpec((_MH, _BA2, GF), lambda i: (0, i, 0)),
            pl.BlockSpec((_MH, _BA2, GF), lambda i: (0, i, 0)),
            pl.BlockSpec((1, GF), lambda i: (0, 0)),
            pl.BlockSpec((1, GF), lambda i: (0, 0)),
        ],
        out_shape=[
            jax.ShapeDtypeStruct((_MH, N, GF), jnp.bfloat16),
            jax.ShapeDtypeStruct((_MH, N, GF), jnp.bfloat16),
            jax.ShapeDtypeStruct((1, GF), jnp.float32),
            jax.ShapeDtypeStruct((1, GF), jnp.float32),
        ],
    )(a, nb3, e_nat, we_t)


def _pass2(g1, g2, sc1, sh1):
    def body(g1_ref, g2_ref, sc_ref, sh_ref, sum_ref, s_ref, q_ref):
        sm = jnp.zeros((_BA2, AF), jnp.float32)
        for g_ref in (g1_ref, g2_ref):
            y = g_ref[...].astype(jnp.float32) * sc_ref[...] + sh_ref[...]
            u = y[..., :AF]
            v = y[..., AF:]
            filt = 0.5 * jnp.tanh(0.5 * u) + 0.5
            core = jnp.where(v > 30.0, v, jnp.log1p(jnp.exp(v)))
            sm = sm + jnp.sum(filt * core, axis=0)
        sum_ref[...] = sm

        mask = _valid_mask(pl.program_id(0))
        smm = jnp.where(mask, sm, 0.0)

        @pl.when(pl.program_id(0) == 0)
        def _():
            s_ref[...] = jnp.zeros_like(s_ref)
            q_ref[...] = jnp.zeros_like(q_ref)

        s_ref[...] += jnp.sum(smm, axis=0).reshape(1, AF)
        q_ref[...] += jnp.sum(smm * smm, axis=0).reshape(1, AF)

    return pl.pallas_call(
        body,
        grid=(_NBLK,),
        in_specs=[
            pl.BlockSpec((_MH, _BA2, GF), lambda i: (0, i, 0)),
            pl.BlockSpec((_MH, _BA2, GF), lambda i: (0, i, 0)),
            pl.BlockSpec((1, GF), lambda i: (0, 0)),
            pl.BlockSpec((1, GF), lambda i: (0, 0)),
        ],
        out_specs=[
            pl.BlockSpec((_BA2, AF), lambda i: (i, 0)),
            pl.BlockSpec((1, AF), lambda i: (0, 0)),
            pl.BlockSpec((1, AF), lambda i: (0, 0)),
        ],
        out_shape=[
            jax.ShapeDtypeStruct((N, AF), jnp.float32),
            jax.ShapeDtypeStruct((1, AF), jnp.float32),
            jax.ShapeDtypeStruct((1, AF), jnp.float32),
        ],
    )(g1, g2, sc1, sh1)


def _pass3(x, sm, sc2, sh2):
    def body(x_ref, sm_ref, sc_ref, sh_ref, o_ref):
        t = x_ref[...] + sm_ref[...] * sc_ref[...] + sh_ref[...]
        o_ref[...] = jax.nn.softplus(t)

    return pl.pallas_call(
        body,
        grid=(N // _BR,),
        in_specs=[
            pl.BlockSpec((_BR, AF), lambda i: (i, 0)),
            pl.BlockSpec((_BR, AF), lambda i: (i, 0)),
            pl.BlockSpec((1, AF), lambda i: (0, 0)),
            pl.BlockSpec((1, AF), lambda i: (0, 0)),
        ],
        out_specs=pl.BlockSpec((_BR, AF), lambda i: (i, 0)),
        out_shape=jax.ShapeDtypeStruct((N, AF), jnp.float32),
    )(x, sm, sc2, sh2)


def _head(x3, wfc_t, bfc, wh_t, bh, wo_t, bo):
    def body(x_ref, wfc_ref, bfc_ref, wh_ref, bh_ref, wo_ref, bo_ref, o_ref):
        crys = jnp.mean(x_ref[...], axis=1)
        crys = jax.nn.relu(
            jnp.dot(crys, wfc_ref[...], preferred_element_type=jnp.float32)
            + bfc_ref[...]
        )
        h = (
            jnp.dot(crys, wh_ref[...], preferred_element_type=jnp.float32)
            + bh_ref[...]
        )
        o_ref[...] = (
            jnp.dot(h, wo_ref[...], preferred_element_type=jnp.float32)
            + bo_ref[...]
        )

    return pl.pallas_call(
        body,
        in_specs=[
            pl.BlockSpec((N0, APC, AF), lambda: (0, 0, 0)),
            pl.BlockSpec((AF, HF), lambda: (0, 0)),
            pl.BlockSpec((1, HF), lambda: (0, 0)),
            pl.BlockSpec((HF, 256), lambda: (0, 0)),
            pl.BlockSpec((1, 256), lambda: (0, 0)),
            pl.BlockSpec((256, 1), lambda: (0, 0)),
            pl.BlockSpec((1, 1), lambda: (0, 0)),
        ],
        out_specs=pl.BlockSpec((N0, 1), lambda: (0, 0)),
        out_shape=jax.ShapeDtypeStruct((N0, 1), jnp.float32),
    )(x3, wfc_t, bfc, wh_t, bh, wo_t, bo)


def kernel(atom_fea, nbr_fea, nbr_fea_idx, crystal_atom_idx,
           W_emb, b_emb, Wf, bf, g1, bb1, g2, bb2,
           W_fc, b_fc, W_h, b_h, W_o, b_o):
    idx_flat = nbr_fea_idx.T.reshape(NM)
    e_nat = jnp.transpose(nbr_fea, (1, 2, 0))
    x = _embed(atom_fea, W_emb.T, b_emb.reshape(1, AF))

    for i in range(NCONV):
        ws_t = Wf[i, :, :AF].T
        wn_t = Wf[i, :, AF:2 * AF].T
        we_t = Wf[i, :, 2 * AF:].T
        bfi = bf[i].reshape(1, GF)

        a, b_tab = _pre(x, ws_t, wn_t, bfi)
        nb = _sc_gather(b_tab, idx_flat)

        gs1, gs2, s1, q1 = _pass1(a, nb.reshape(M, N, GF), e_nat, we_t)
        mean1 = s1 / NM
        var1 = q1 / NM - mean1 * mean1
        sc1 = g1[i].reshape(1, GF) / jnp.sqrt(var1 + 1e-5)
        sh1 = bb1[i].reshape(1, GF) - mean1 * sc1

        sm, s2, q2 = _pass2(gs1, gs2, sc1, sh1)
        mean2 = s2 / N
        var2 = q2 / N - mean2 * mean2
        sc2 = g2[i].reshape(1, AF) / jnp.sqrt(var2 + 1e-5)
        sh2 = bb2[i].reshape(1, AF) - mean2 * sc2

        x = _pass3(x, sm, sc2, sh2)

    out = _head(x.reshape(N0, APC, AF), W_fc.T, b_fc.reshape(1, HF),
                W_h.T, b_h.reshape(1, 256), W_o.T, b_o.reshape(1, 1))
    return (out,)


# speedup vs baseline: 1.0415x; 1.0167x over previous
import functools

import jax
import jax.numpy as jnp
from jax import lax
from jax.experimental import pallas as pl
from jax.experimental.pallas import tpu as pltpu
from jax.experimental.pallas import tpu_sc as plsc

N = 50000
M = 16
ORIG = 92
NBR = 16
AF = 64
HF = 128
NCONV = 3
N0 = 500
APC = 100
NM = N * M
GF = 2 * AF


_CHI = 1000
_CG0 = 504
_CG1 = 496


def _sc_gather(table, idx_flat):
    info = plsc.get_sparse_core_info()
    nc, ns = info.num_cores, info.num_subcores
    nw = nc * ns
    epw = NM // nw
    nsup = epw // _CHI

    mesh = plsc.VectorSubcoreMesh(core_axis_name="c", subcore_axis_name="s")

    @functools.partial(
        pl.kernel,
        mesh=mesh,
        out_type=jax.ShapeDtypeStruct((NM, GF), jnp.float32),
        scratch_types=[
            pltpu.VMEM((_CHI,), jnp.int32),
            pltpu.VMEM((_CHI,), jnp.int32),
            pltpu.VMEM((_CG0, GF), jnp.float32),
            pltpu.VMEM((_CG1, GF), jnp.float32),
            pltpu.SemaphoreType.DMA,
            pltpu.SemaphoreType.DMA,
        ],
    )
    def gather_k(tab_hbm, idx_hbm, out_hbm, idx0, idx1, rows0, rows1,
                 sem0, sem1):
        wid = lax.axis_index("s") * nc + lax.axis_index("c")
        base = wid * epw
        idxs = (idx0, idx1)

        def gstart0(iv):
            return pltpu.async_copy(tab_hbm.at[iv.at[pl.ds(0, _CG0)]],
                                    rows0, sem0)

        def gstart1(iv):
            return pltpu.async_copy(tab_hbm.at[iv.at[pl.ds(_CG0, _CG1)]],
                                    rows1, sem1)

        pltpu.sync_copy(idx_hbm.at[pl.ds(base, _CHI)], idx0)
        cp = gstart0(idx0)
        for c in range(nsup):
            iv = idxs[c % 2]
            cp2 = gstart1(iv)
            cp.wait()
            pltpu.sync_copy(rows0, out_hbm.at[pl.ds(base + c * _CHI, _CG0)])
            if c + 1 < nsup:
                nxt = idxs[(c + 1) % 2]
                pltpu.sync_copy(
                    idx_hbm.at[pl.ds(base + (c + 1) * _CHI, _CHI)], nxt)
                cp = gstart0(nxt)
            cp2.wait()
            pltpu.sync_copy(
                rows1, out_hbm.at[pl.ds(base + c * _CHI + _CG0, _CG1)])

    return gather_k(table, idx_flat)


_BR = 2000
_BA = 1000
_BE = _BA * M


def _embed(atom_fea, w_t, b):
    def body(a_ref, w_ref, b_ref, o_ref):
        o_ref[...] = (
            jnp.dot(a_ref[...], w_ref[...], preferred_element_type=jnp.float32)
            + b_ref[...]
        )

    return pl.pallas_call(
        body,
        grid=(N // _BR,),
        in_specs=[
            pl.BlockSpec((_BR, ORIG), lambda i: (i, 0)),
            pl.BlockSpec((ORIG, AF), lambda i: (0, 0)),
            pl.BlockSpec((1, AF), lambda i: (0, 0)),
        ],
        out_specs=pl.BlockSpec((_BR, AF), lambda i: (i, 0)),
        out_shape=jax.ShapeDtypeStruct((N, AF), jnp.float32),
    )(atom_fea, w_t, b)


def _softplus(t):
    return jnp.where(t > 30.0, t, jnp.log1p(jnp.exp(t)))


def _pre_fused(xp, sm, sc2, sh2, ws_t, wn_t, bfi):
    def body(xp_ref, sm_ref, sc_ref, sh_ref, ws_ref, wn_ref, bf_ref,
             x_ref, a_ref, b_ref):
        xb = _softplus(xp_ref[...] + sm_ref[...] * sc_ref[...] + sh_ref[...])
        x_ref[...] = xb
        a_ref[...] = (
            jnp.dot(xb, ws_ref[...], preferred_element_type=jnp.float32)
            + bf_ref[...]
        )
        b_ref[...] = jnp.dot(xb, wn_ref[...], preferred_element_type=jnp.float32)

    return pl.pallas_call(
        body,
        grid=(N // _BR,),
        in_specs=[
            pl.BlockSpec((_BR, AF), lambda i: (i, 0)),
            pl.BlockSpec((_BR, AF), lambda i: (i, 0)),
            pl.BlockSpec((1, AF), lambda i: (0, 0)),
            pl.BlockSpec((1, AF), lambda i: (0, 0)),
            pl.BlockSpec((AF, GF), lambda i: (0, 0)),
            pl.BlockSpec((AF, GF), lambda i: (0, 0)),
            pl.BlockSpec((1, GF), lambda i: (0, 0)),
        ],
        out_specs=[
            pl.BlockSpec((_BR, AF), lambda i: (i, 0)),
            pl.BlockSpec((_BR, GF), lambda i: (i, 0)),
            pl.BlockSpec((_BR, GF), lambda i: (i, 0)),
        ],
        out_shape=[
            jax.ShapeDtypeStruct((N, AF), jnp.float32),
            jax.ShapeDtypeStruct((N, GF), jnp.float32),
            jax.ShapeDtypeStruct((N, GF), jnp.float32),
        ],
    )(xp, sm, sc2, sh2, ws_t, wn_t, bfi)


def _pre(x, ws_t, wn_t, bfi):
    def body(x_ref, ws_ref, wn_ref, bf_ref, a_ref, b_ref):
        xb = x_ref[...]
        a_ref[...] = (
            jnp.dot(xb, ws_ref[...], preferred_element_type=jnp.float32)
            + bf_ref[...]
        )
        b_ref[...] = jnp.dot(xb, wn_ref[...], preferred_element_type=jnp.float32)

    return pl.pallas_call(
        body,
        grid=(N // _BR,),
        in_specs=[
            pl.BlockSpec((_BR, AF), lambda i: (i, 0)),
            pl.BlockSpec((AF, GF), lambda i: (0, 0)),
            pl.BlockSpec((AF, GF), lambda i: (0, 0)),
            pl.BlockSpec((1, GF), lambda i: (0, 0)),
        ],
        out_specs=[
            pl.BlockSpec((_BR, GF), lambda i: (i, 0)),
            pl.BlockSpec((_BR, GF), lambda i: (i, 0)),
        ],
        out_shape=[
            jax.ShapeDtypeStruct((N, GF), jnp.float32),
            jax.ShapeDtypeStruct((N, GF), jnp.float32),
        ],
    )(x, ws_t, wn_t, bfi)


_BA2 = 1024
_NBLK = -(-N // _BA2)


def _valid_mask(pid):
    row = lax.broadcasted_iota(jnp.int32, (_BA2, 1), 0)
    return row < (N - pid * _BA2)


_MH = M // 2


def _pass1(a, nb3, e_nat, we_t):
    def body(a_ref, nb_ref, e_ref, we_ref, g0_ref, g1_ref, s_ref, q_ref):
        a2 = a_ref[...]
        s_rows = jnp.zeros((_BA2, GF), jnp.float32)
        q_rows = jnp.zeros((_BA2, GF), jnp.float32)
        for m in range(M):
            ecm = lax.dot_general(
                e_ref[m], we_ref[...],
                dimension_numbers=(((0,), (0,)), ((), ())),
                preferred_element_type=jnp.float32,
            )
            gm = nb_ref[m] + ecm + a2
            if m < _MH:
                g0_ref[m] = gm.astype(jnp.bfloat16)
            else:
                g1_ref[m - _MH] = gm.astype(jnp.bfloat16)
            s_rows = s_rows + gm
            q_rows = q_rows + gm * gm

        mask = _valid_mask(pl.program_id(0))
        s_rows = jnp.where(mask, s_rows, 0.0)
        q_rows = jnp.where(mask, q_rows, 0.0)

        @pl.when(pl.program_id(0) == 0)
        def _():
            s_ref[...] = jnp.zeros_like(s_ref)
            q_ref[...] = jnp.zeros_like(q_ref)

        s_ref[...] += jnp.sum(s_rows, axis=0).reshape(1, GF)
        q_ref[...] += jnp.sum(q_rows, axis=0).reshape(1, GF)

    return pl.pallas_call(
        body,
        grid=(_NBLK,),
        in_specs=[
            pl.BlockSpec((_BA2, GF), lambda i: (i, 0)),
            pl.BlockSpec((M, _BA2, GF), lambda i: (0, i, 0)),
            pl.BlockSpec((M, NBR, _BA2), lambda i: (0, 0, i)),
            pl.BlockSpec((NBR, GF), lambda i: (0, 0)),
        ],
        out_specs=[
            pl.BlockSpec((_MH, _BA2, GF), lambda i: (0, i, 0)),
            pl.BlockSpec((_MH, _BA2, GF), lambda i: (0, i, 0)),
            pl.BlockSpec((1, GF), lambda i: (0, 0)),
            pl.BlockSpec((1, GF), lambda i: (0, 0)),
        ],
        out_shape=[
            jax.ShapeDtypeStruct((_MH, N, GF), jnp.bfloat16),
            jax.ShapeDtypeStruct((_MH, N, GF), jnp.bfloat16),
            jax.ShapeDtypeStruct((1, GF), jnp.float32),
            jax.ShapeDtypeStruct((1, GF), jnp.float32),
        ],
    )(a, nb3, e_nat, we_t)


def _pass2(g1, g2, sc1, sh1):
    def body(g1_ref, g2_ref, sc_ref, sh_ref, sum_ref, s_ref, q_ref):
        sm = jnp.zeros((_BA2, AF), jnp.float32)
        for g_ref in (g1_ref, g2_ref):
            y = g_ref[...].astype(jnp.float32) * sc_ref[...] + sh_ref[...]
            u = y[..., :AF]
            v = y[..., AF:]
            filt = 0.5 * jnp.tanh(0.5 * u) + 0.5
            core = jnp.where(v > 30.0, v, jnp.log1p(jnp.exp(v)))
            sm = sm + jnp.sum(filt * core, axis=0)
        sum_ref[...] = sm

        mask = _valid_mask(pl.program_id(0))
        smm = jnp.where(mask, sm, 0.0)

        @pl.when(pl.program_id(0) == 0)
        def _():
            s_ref[...] = jnp.zeros_like(s_ref)
            q_ref[...] = jnp.zeros_like(q_ref)

        s_ref[...] += jnp.sum(smm, axis=0).reshape(1, AF)
        q_ref[...] += jnp.sum(smm * smm, axis=0).reshape(1, AF)

    return pl.pallas_call(
        body,
        grid=(_NBLK,),
        in_specs=[
            pl.BlockSpec((_MH, _BA2, GF), lambda i: (0, i, 0)),
            pl.BlockSpec((_MH, _BA2, GF), lambda i: (0, i, 0)),
            pl.BlockSpec((1, GF), lambda i: (0, 0)),
            pl.BlockSpec((1, GF), lambda i: (0, 0)),
        ],
        out_specs=[
            pl.BlockSpec((_BA2, AF), lambda i: (i, 0)),
            pl.BlockSpec((1, AF), lambda i: (0, 0)),
            pl.BlockSpec((1, AF), lambda i: (0, 0)),
        ],
        out_shape=[
            jax.ShapeDtypeStruct((N, AF), jnp.float32),
            jax.ShapeDtypeStruct((1, AF), jnp.float32),
            jax.ShapeDtypeStruct((1, AF), jnp.float32),
        ],
    )(g1, g2, sc1, sh1)


def _pass3(x, sm, sc2, sh2):
    def body(x_ref, sm_ref, sc_ref, sh_ref, o_ref):
        t = x_ref[...] + sm_ref[...] * sc_ref[...] + sh_ref[...]
        o_ref[...] = jax.nn.softplus(t)

    return pl.pallas_call(
        body,
        grid=(N // _BR,),
        in_specs=[
            pl.BlockSpec((_BR, AF), lambda i: (i, 0)),
            pl.BlockSpec((_BR, AF), lambda i: (i, 0)),
            pl.BlockSpec((1, AF), lambda i: (0, 0)),
            pl.BlockSpec((1, AF), lambda i: (0, 0)),
        ],
        out_specs=pl.BlockSpec((_BR, AF), lambda i: (i, 0)),
        out_shape=jax.ShapeDtypeStruct((N, AF), jnp.float32),
    )(x, sm, sc2, sh2)


def _head(x3, wfc_t, bfc, wh_t, bh, wo_t, bo):
    def body(x_ref, wfc_ref, bfc_ref, wh_ref, bh_ref, wo_ref, bo_ref, o_ref):
        crys = jnp.mean(x_ref[...], axis=1)
        crys = jax.nn.relu(
            jnp.dot(crys, wfc_ref[...], preferred_element_type=jnp.float32)
            + bfc_ref[...]
        )
        h = (
            jnp.dot(crys, wh_ref[...], preferred_element_type=jnp.float32)
            + bh_ref[...]
        )
        o_ref[...] = (
            jnp.dot(h, wo_ref[...], preferred_element_type=jnp.float32)
            + bo_ref[...]
        )

    return pl.pallas_call(
        body,
        in_specs=[
            pl.BlockSpec((N0, APC, AF), lambda: (0, 0, 0)),
            pl.BlockSpec((AF, HF), lambda: (0, 0)),
            pl.BlockSpec((1, HF), lambda: (0, 0)),
            pl.BlockSpec((HF, 256), lambda: (0, 0)),
            pl.BlockSpec((1, 256), lambda: (0, 0)),
            pl.BlockSpec((256, 1), lambda: (0, 0)),
            pl.BlockSpec((1, 1), lambda: (0, 0)),
        ],
        out_specs=pl.BlockSpec((N0, 1), lambda: (0, 0)),
        out_shape=jax.ShapeDtypeStruct((N0, 1), jnp.float32),
    )(x3, wfc_t, bfc, wh_t, bh, wo_t, bo)


def kernel(atom_fea, nbr_fea, nbr_fea_idx, crystal_atom_idx,
           W_emb, b_emb, Wf, bf, g1, bb1, g2, bb2,
           W_fc, b_fc, W_h, b_h, W_o, b_o):
    idx_flat = nbr_fea_idx.T.reshape(NM)
    e_nat = jnp.transpose(nbr_fea, (1, 2, 0))
    x = _embed(atom_fea, W_emb.T, b_emb.reshape(1, AF))

    pend = None
    for i in range(NCONV):
        ws_t = Wf[i, :, :AF].T
        wn_t = Wf[i, :, AF:2 * AF].T
        we_t = Wf[i, :, 2 * AF:].T
        bfi = bf[i].reshape(1, GF)

        if pend is None:
            a, b_tab = _pre(x, ws_t, wn_t, bfi)
        else:
            x, a, b_tab = _pre_fused(x, *pend, ws_t, wn_t, bfi)
        nb = _sc_gather(b_tab, idx_flat)

        gs1, gs2, s1, q1 = _pass1(a, nb.reshape(M, N, GF), e_nat, we_t)
        mean1 = s1 / NM
        var1 = q1 / NM - mean1 * mean1
        sc1 = g1[i].reshape(1, GF) / jnp.sqrt(var1 + 1e-5)
        sh1 = bb1[i].reshape(1, GF) - mean1 * sc1

        sm, s2, q2 = _pass2(gs1, gs2, sc1, sh1)
        mean2 = s2 / N
        var2 = q2 / N - mean2 * mean2
        sc2 = g2[i].reshape(1, AF) / jnp.sqrt(var2 + 1e-5)
        sh2 = bb2[i].reshape(1, AF) - mean2 * sc2
        pend = (sm, sc2, sh2)

    x = _pass3(x, *pend)

    out = _head(x.reshape(N0, APC, AF), W_fc.T, b_fc.reshape(1, HF),
                W_h.T, b_h.reshape(1, 256), W_o.T, b_o.reshape(1, 1))
    return (out,)
